# baseline (speedup 1.0000x reference)
import functools

import jax
import jax.numpy as jnp
from jax import lax
from jax.experimental import pallas as pl
from jax.experimental.pallas import tpu as pltpu
from jax.experimental.pallas import tpu_sc as plsc

ALPHA = 0.1
K = 3
NC = 2
NS = 16
B = 128
QR = 5120
NBUF = 2


def _sc_hop_kernel(np_, d, nb2):
  qch = QR // NS
  ng = nb2 // NBUF
  mesh = plsc.VectorSubcoreMesh(core_axis_name="c", subcore_axis_name="s")

  @functools.partial(
      pl.kernel,
      out_type=jax.ShapeDtypeStruct((NC, QR, d), jnp.float32),
      mesh=mesh,
      scratch_types=[
          pltpu.VMEM((nb2, B), jnp.int32),
          pltpu.VMEM((nb2, B), jnp.int32),
          pltpu.VMEM((NBUF, B), jnp.int32),
          *[pltpu.VMEM((B, d), jnp.float32) for _ in range(NBUF)],
          pltpu.VMEM((64, d), jnp.float32),
          pltpu.VMEM_SHARED((QR + 8, d), jnp.float32),
          *[pltpu.SemaphoreType.DMA for _ in range(2 * NBUF)],
      ],
  )
  def k(y_hbm, src_hbm, dst_hbm, zeros_hbm, out_hbm, src_v, gdst_v, scidx_v,
        *rest):
    rows = rest[:NBUF]
    zbuf_v = rest[NBUF]
    accum = rest[NBUF + 1]
    gsem = rest[NBUF + 2:2 * NBUF + 2]
    ssem = rest[2 * NBUF + 2:]
    c = lax.axis_index("c")
    s = lax.axis_index("s")
    cbase = c * QR
    pltpu.sync_copy(zeros_hbm, zbuf_v)
    for z in range(qch // 64):
      pltpu.sync_copy(zbuf_v, accum.at[pl.ds(s * qch + z * 64, 64)])
    pltpu.sync_copy(src_hbm.at[s], src_v)
    pltpu.sync_copy(dst_hbm.at[s], gdst_v)

    def remap(j, carry):
      for kk in range(B // 16):
        sl = pl.ds(kk * 16, 16)
        gdst_v[j, sl] = gdst_v[j, sl] - cbase
      return carry

    lax.fori_loop(0, nb2, remap, 0)
    plsc.subcore_barrier()
    for b in range(NBUF):
      pltpu.async_copy(y_hbm.at[src_v.at[b]], rows[b], gsem[b])

    def group(g, carry):
      for b in range(NBUF):
        j = g * NBUF + b
        pltpu.make_async_copy(y_hbm.at[src_v.at[j]], rows[b],
                              gsem[b]).wait()
        for kk in range(B // 16):
          sl = pl.ds(kk * 16, 16)
          v = gdst_v[j, sl]
          ok = (v >= 0) & (v < QR)
          scidx_v[b, sl] = jnp.where(ok, v, QR)
        pltpu.async_copy(rows[b], accum.at[scidx_v.at[b]], ssem[b],
                         add=True)
      for b in range(NBUF):
        pltpu.make_async_copy(rows[b], accum.at[scidx_v.at[b]],
                              ssem[b]).wait()

        @pl.when(g < ng - 1)
        def _():
          jn = (g + 1) * NBUF + b
          pltpu.async_copy(y_hbm.at[src_v.at[jn]], rows[b], gsem[b])

      return carry

    lax.fori_loop(0, ng, group, 0)
    plsc.subcore_barrier()
    for z in range(qch // 64):
      pltpu.sync_copy(accum.at[pl.ds(s * qch + z * 64, 64)], zbuf_v)
      pltpu.sync_copy(zbuf_v, out_hbm.at[c, pl.ds(s * qch + z * 64, 64)])

  return k


def _tc_prep(deg, x0):
  np_, d = x0.shape
  br = 1024

  def body(deg_ref, x0_ref, y_ref):
    r = lax.rsqrt(1.0 + deg_ref[...])
    y_ref[...] = r * x0_ref[...]

  spec = pl.BlockSpec((br, d), lambda i: (i, 0))
  return pl.pallas_call(
      body,
      grid=(np_ // br,),
      in_specs=[spec, spec],
      out_specs=spec,
      out_shape=jax.ShapeDtypeStruct((np_, d), jnp.float32),
  )(deg, x0)


def _tc_combine(deg, agg, y, x0):
  np_, d = x0.shape
  br = 1024

  def body(deg_ref, agg_ref, y_ref, x0_ref, x_ref, yn_ref):
    r = lax.rsqrt(1.0 + deg_ref[...])
    x = (1.0 - ALPHA) * r * (agg_ref[...] + y_ref[...]) + ALPHA * x0_ref[...]
    x_ref[...] = x
    yn_ref[...] = r * x

  spec = pl.BlockSpec((br, d), lambda i: (i, 0))
  return pl.pallas_call(
      body,
      grid=(np_ // br,),
      in_specs=[spec, spec, spec, spec],
      out_specs=[spec, spec],
      out_shape=[
          jax.ShapeDtypeStruct((np_, d), jnp.float32),
          jax.ShapeDtypeStruct((np_, d), jnp.float32),
      ],
  )(deg, agg, y, x0)


@jax.jit
def kernel(features, edge_index):
  n, d = features.shape
  e = edge_index.shape[1]

  np_ = ((n + 1 + 2047) // 2048) * 2048
  nb2 = -(-e // (NS * B))
  nb2 = ((nb2 + 2 * NBUF - 1) // (2 * NBUF)) * (2 * NBUF)
  epad = NS * nb2 * B
  pad = epad - e

  src = jnp.concatenate(
      [edge_index[0], jnp.full((pad,), n, dtype=jnp.int32)]
  ).reshape(NS, nb2, B)
  dst = jnp.concatenate(
      [edge_index[1], jnp.full((pad,), n, dtype=jnp.int32)]
  ).reshape(NS, nb2, B)

  x0 = jnp.zeros((np_, d), jnp.float32).at[:n].set(features)
  onesb = jnp.ones((B, d), jnp.float32)
  zerosb = jnp.zeros((64, d), jnp.float32)

  hop = _sc_hop_kernel(np_, d, nb2)
  onesf = jnp.ones((np_, d), jnp.float32)
  deg = hop(onesf, src, dst, zerosb).reshape(np_, d)
  y = _tc_prep(deg, x0)
  x = x0
  for _ in range(K):
    agg = hop(y, src, dst, zerosb).reshape(np_, d)
    x, y = _tc_combine(deg, agg, y, x0)
  return x[:n]


# speedup vs baseline: 1.5104x; 1.5104x over previous
import functools

import jax
import jax.numpy as jnp
from jax import lax
from jax.experimental import pallas as pl
from jax.experimental.pallas import tpu as pltpu
from jax.experimental.pallas import tpu_sc as plsc

ALPHA = 0.1
K = 3
NC = 2
NS = 16
B = 128
QR = 5120
NBUF = 2


def _sc_hop_kernel(np_, d, nb2):
  qch = QR // NS
  ng = nb2 // NBUF
  mesh = plsc.VectorSubcoreMesh(core_axis_name="c", subcore_axis_name="s")

  @functools.partial(
      pl.kernel,
      out_type=jax.ShapeDtypeStruct((NC, QR, d), jnp.float32),
      mesh=mesh,
      scratch_types=[
          pltpu.VMEM((nb2, B), jnp.int32),
          pltpu.VMEM((nb2, B), jnp.int32),
          pltpu.VMEM((NBUF, B), jnp.int32),
          *[pltpu.VMEM((B, d), jnp.float32) for _ in range(NBUF)],
          pltpu.VMEM((64, d), jnp.float32),
          pltpu.VMEM_SHARED((QR + 8, d), jnp.float32),
          *[pltpu.SemaphoreType.DMA for _ in range(2 * NBUF)],
      ],
  )
  def k(y_hbm, src_hbm, dst_hbm, zeros_hbm, out_hbm, src_v, gdst_v, scidx_v,
        *rest):
    rows = rest[:NBUF]
    zbuf_v = rest[NBUF]
    accum = rest[NBUF + 1]
    gsem = rest[NBUF + 2:2 * NBUF + 2]
    ssem = rest[2 * NBUF + 2:]
    c = lax.axis_index("c")
    s = lax.axis_index("s")
    cbase = c * QR
    pltpu.sync_copy(zeros_hbm, zbuf_v)
    for z in range(qch // 64):
      pltpu.sync_copy(zbuf_v, accum.at[pl.ds(s * qch + z * 64, 64)])
    pltpu.sync_copy(src_hbm.at[s], src_v)
    pltpu.sync_copy(dst_hbm.at[s], gdst_v)

    def remap(j, carry):
      for kk in range(B // 16):
        sl = pl.ds(kk * 16, 16)
        gdst_v[j, sl] = gdst_v[j, sl] - cbase
      return carry

    lax.fori_loop(0, nb2, remap, 0)
    plsc.subcore_barrier()
    for b in range(NBUF):
      pltpu.async_copy(y_hbm.at[src_v.at[b]], rows[b], gsem[b])

    def group(g, carry):
      for b in range(NBUF):
        j = g * NBUF + b
        pltpu.make_async_copy(y_hbm.at[src_v.at[j]], rows[b],
                              gsem[b]).wait()
        for kk in range(B // 16):
          sl = pl.ds(kk * 16, 16)
          v = gdst_v[j, sl]
          ok = (v >= 0) & (v < QR)
          scidx_v[b, sl] = jnp.where(ok, v, QR)
        pltpu.async_copy(rows[b], accum.at[scidx_v.at[b]], ssem[b],
                         add=True)
      for b in range(NBUF):
        pltpu.make_async_copy(rows[b], accum.at[scidx_v.at[b]],
                              ssem[b]).wait()

        @pl.when(g < ng - 1)
        def _():
          jn = (g + 1) * NBUF + b
          pltpu.async_copy(y_hbm.at[src_v.at[jn]], rows[b], gsem[b])

      return carry

    lax.fori_loop(0, ng, group, 0)
    plsc.subcore_barrier()
    for z in range(qch // 64):
      pltpu.sync_copy(accum.at[pl.ds(s * qch + z * 64, 64)], zbuf_v)
      pltpu.sync_copy(zbuf_v, out_hbm.at[c, pl.ds(s * qch + z * 64, 64)])

  return k


def _tc_prep(deg, x0):
  np_, d = x0.shape
  br = 1024

  def body(deg_ref, x0_ref, y_ref):
    r = lax.rsqrt(1.0 + deg_ref[...])
    y_ref[...] = r * x0_ref[...]

  spec = pl.BlockSpec((br, d), lambda i: (i, 0))
  return pl.pallas_call(
      body,
      grid=(np_ // br,),
      in_specs=[spec, spec],
      out_specs=spec,
      out_shape=jax.ShapeDtypeStruct((np_, d), jnp.float32),
  )(deg, x0)


def _tc_combine(deg, agg, y, x0):
  np_, d = x0.shape
  br = 1024

  def body(deg_ref, agg_ref, y_ref, x0_ref, x_ref, yn_ref):
    r = lax.rsqrt(1.0 + deg_ref[...])
    x = (1.0 - ALPHA) * r * (agg_ref[...] + y_ref[...]) + ALPHA * x0_ref[...]
    x_ref[...] = x
    yn_ref[...] = r * x

  spec = pl.BlockSpec((br, d), lambda i: (i, 0))
  return pl.pallas_call(
      body,
      grid=(np_ // br,),
      in_specs=[spec, spec, spec, spec],
      out_specs=[spec, spec],
      out_shape=[
          jax.ShapeDtypeStruct((np_, d), jnp.float32),
          jax.ShapeDtypeStruct((np_, d), jnp.float32),
      ],
  )(deg, agg, y, x0)


@jax.jit
def kernel(features, edge_index):
  n, d = features.shape
  e = edge_index.shape[1]

  np_ = ((n + 1 + 2047) // 2048) * 2048
  nb2 = -(-e // (NS * B))
  nb2 = ((nb2 + NBUF - 1) // NBUF) * NBUF
  epad = NS * nb2 * B
  pad = epad - e

  src = jnp.concatenate(
      [edge_index[0], jnp.full((pad,), n, dtype=jnp.int32)]
  ).reshape(NS, nb2, B)
  dst = jnp.concatenate(
      [edge_index[1], jnp.full((pad,), n, dtype=jnp.int32)]
  ).reshape(NS, nb2, B)

  x0 = jnp.zeros((np_, d), jnp.float32).at[:n].set(features)
  onesb = jnp.ones((B, d), jnp.float32)
  zerosb = jnp.zeros((64, d), jnp.float32)

  hop = _sc_hop_kernel(np_, d, nb2)
  onesf = jnp.ones((np_, d), jnp.float32)
  deg = hop(onesf, src, dst, zerosb).reshape(np_, d)
  y = _tc_prep(deg, x0)
  x = x0
  for _ in range(K):
    agg = hop(y, src, dst, zerosb).reshape(np_, d)
    x, y = _tc_combine(deg, agg, y, x0)
  return x[:n]


# speedup vs baseline: 2.6618x; 1.7623x over previous
import functools

import jax
import jax.numpy as jnp
from jax import lax
from jax.experimental import pallas as pl
from jax.experimental.pallas import tpu as pltpu
from jax.experimental.pallas import tpu_sc as plsc

ALPHA = 0.1
K = 3
NC = 2
NS = 16
B = 128
QR = 5120
NBUF = 2


def _sc_hop_kernel(np_, d, nb2):
  qch = QR // NS
  ng = nb2 // NBUF
  mesh = plsc.VectorSubcoreMesh(core_axis_name="c", subcore_axis_name="s")

  @functools.partial(
      pl.kernel,
      out_type=jax.ShapeDtypeStruct((NC, QR, d), jnp.float32),
      mesh=mesh,
      scratch_types=[
          pltpu.VMEM((nb2, B), jnp.int32),
          pltpu.VMEM((nb2, B), jnp.int32),
          pltpu.VMEM((NBUF, B), jnp.int32),
          *[pltpu.VMEM((B, d), jnp.float32) for _ in range(NBUF)],
          pltpu.VMEM((64, d), jnp.float32),
          pltpu.VMEM_SHARED((QR + 8, d), jnp.float32),
          *[pltpu.SemaphoreType.DMA for _ in range(2 * NBUF)],
      ],
  )
  def k(y_hbm, src_hbm, dst_hbm, zeros_hbm, out_hbm, src_v, gdst_v, scidx_v,
        *rest):
    rows = rest[:NBUF]
    zbuf_v = rest[NBUF]
    accum = rest[NBUF + 1]
    gsem = rest[NBUF + 2:2 * NBUF + 2]
    ssem = rest[2 * NBUF + 2:]
    c = lax.axis_index("c")
    s = lax.axis_index("s")
    cbase = c * QR
    pltpu.sync_copy(zeros_hbm, zbuf_v)
    for z in range(qch // 64):
      pltpu.sync_copy(zbuf_v, accum.at[pl.ds(s * qch + z * 64, 64)])
    pltpu.sync_copy(src_hbm.at[s], src_v)
    pltpu.sync_copy(dst_hbm.at[s], gdst_v)

    def remap(j, carry):
      for kk in range(B // 16):
        sl = pl.ds(kk * 16, 16)
        gdst_v[j, sl] = gdst_v[j, sl] - cbase
      return carry

    lax.fori_loop(0, nb2, remap, 0)
    plsc.subcore_barrier()
    for b in range(NBUF):
      pltpu.async_copy(y_hbm.at[src_v.at[b]], rows[b], gsem[b])

    def group(g, carry):
      for b in range(NBUF):
        j = g * NBUF + b
        pltpu.make_async_copy(y_hbm.at[src_v.at[j]], rows[b],
                              gsem[b]).wait()
        for kk in range(B // 16):
          sl = pl.ds(kk * 16, 16)
          v = gdst_v[j, sl]
          ok = (v >= 0) & (v < QR)
          scidx_v[b, sl] = jnp.where(ok, v, QR)
        pltpu.async_copy(rows[b], accum.at[scidx_v.at[b]], ssem[b],
                         add=True)
      for b in range(NBUF):
        pltpu.make_async_copy(rows[b], accum.at[scidx_v.at[b]],
                              ssem[b]).wait()

        @pl.when(g < ng - 1)
        def _():
          jn = (g + 1) * NBUF + b
          pltpu.async_copy(y_hbm.at[src_v.at[jn]], rows[b], gsem[b])

      return carry

    lax.fori_loop(0, ng, group, 0)
    plsc.subcore_barrier()
    for z in range(qch // 64):
      pltpu.sync_copy(accum.at[pl.ds(s * qch + z * 64, 64)], zbuf_v)
      pltpu.sync_copy(zbuf_v, out_hbm.at[c, pl.ds(s * qch + z * 64, 64)])

  return k


def _tc_prep(deg, x0):
  np_, d = x0.shape
  br = 1024

  def body(deg_ref, x0_ref, y_ref):
    r = lax.rsqrt(1.0 + deg_ref[...])
    y_ref[...] = r * x0_ref[...]

  spec = pl.BlockSpec((br, d), lambda i: (i, 0))
  return pl.pallas_call(
      body,
      grid=(np_ // br,),
      in_specs=[spec, spec],
      out_specs=spec,
      out_shape=jax.ShapeDtypeStruct((np_, d), jnp.float32),
  )(deg, x0)


def _tc_combine(deg, agg, y, x0):
  np_, d = x0.shape
  br = 1024

  def body(deg_ref, agg_ref, y_ref, x0_ref, x_ref, yn_ref):
    r = lax.rsqrt(1.0 + deg_ref[...])
    x = (1.0 - ALPHA) * r * (agg_ref[...] + y_ref[...]) + ALPHA * x0_ref[...]
    x_ref[...] = x
    yn_ref[...] = r * x

  spec = pl.BlockSpec((br, d), lambda i: (i, 0))
  return pl.pallas_call(
      body,
      grid=(np_ // br,),
      in_specs=[spec, spec, spec, spec],
      out_specs=[spec, spec],
      out_shape=[
          jax.ShapeDtypeStruct((np_, d), jnp.float32),
          jax.ShapeDtypeStruct((np_, d), jnp.float32),
      ],
  )(deg, agg, y, x0)


@jax.jit
def kernel(features, edge_index):
  n, d = features.shape
  e = edge_index.shape[1]

  np_ = ((n + 1 + 2047) // 2048) * 2048
  nb2 = -(-e // (NS * B))
  nb2 = ((nb2 + NBUF - 1) // NBUF) * NBUF
  epad = NS * nb2 * B
  pad = epad - e

  deadpad = n + jnp.arange(pad, dtype=jnp.int32) % (np_ - n)
  src = jnp.concatenate([edge_index[0], deadpad]).reshape(NS, nb2, B)
  dst = jnp.concatenate([edge_index[1], deadpad]).reshape(NS, nb2, B)

  x0 = jnp.zeros((np_, d), jnp.float32).at[:n].set(features)
  onesb = jnp.ones((B, d), jnp.float32)
  zerosb = jnp.zeros((64, d), jnp.float32)

  hop = _sc_hop_kernel(np_, d, nb2)
  onesf = jnp.ones((np_, d), jnp.float32)
  deg = hop(onesf, src, dst, zerosb).reshape(np_, d)
  y = _tc_prep(deg, x0)
  x = x0
  for _ in range(K):
    agg = hop(y, src, dst, zerosb).reshape(np_, d)
    x, y = _tc_combine(deg, agg, y, x0)
  return x[:n]
